# baseline (speedup 1.0000x reference)
import functools

import numpy as np
import jax
import jax.numpy as jnp
from jax import lax
from jax.experimental import pallas as pl
from jax.experimental.pallas import tpu as pltpu
from jax.experimental.pallas import tpu_sc as plsc

NN = 10000
NE = 320000
DH = 128
CP = 4
BB = 64
EPS = 1e-8

NC, NS = 2, 16
NW = NC * NS
EPW = NE // NW
CH = 80
NCH = EPW // CH
RPT = 624
RPT_LAST = NN - (NS - 1) * RPT
NNCP = NN * CP
NNCP_PAD = 40960

@functools.lru_cache(maxsize=1)
def _sc_mesh():
    return plsc.VectorSubcoreMesh(core_axis_name="c", subcore_axis_name="s",
                                  num_cores=NC, num_subcores=NS)

_EP = 2048
_r2, _c2 = np.triu_indices(BB, k=1)
_E2 = _r2.shape[0]
_R_NP = np.zeros((_EP, BB), np.float32)
_C_NP = np.zeros((_EP, BB), np.float32)
_R_NP[np.arange(_E2), _r2] = 1.0
_C_NP[np.arange(_E2), _c2] = 1.0
_CNT2_NP = np.maximum(np.bincount(_r2, minlength=BB), 1).astype(np.float32)[:, None]

_PERM_NP = np.empty((DH,), np.int64)
for _g in range(DH // 32):
    for _i in range(16):
        _PERM_NP[_g * 32 + 2 * _i] = _g * 32 + _i
        _PERM_NP[_g * 32 + 2 * _i + 1] = _g * 32 + 16 + _i


def _sc_gather(h, coordp, rows3, cols3, with_cd=True):
    DW = DH // 2
    out_type = [jax.ShapeDtypeStruct((NE * DW,), jnp.int32),
                jax.ShapeDtypeStruct((NE * DW,), jnp.int32)]
    scratch = [pltpu.VMEM((NCH, CH), jnp.int32),
               pltpu.VMEM((NCH, CH), jnp.int32),
               pltpu.VMEM((2, CH, DH), jnp.float32),
               pltpu.VMEM((2, CH, DH), jnp.float32),
               pltpu.VMEM((CH * DW,), jnp.int32),
               pltpu.VMEM((CH * DW,), jnp.int32)]
    if with_cd:
        out_type += [jax.ShapeDtypeStruct((NE * CP,), jnp.float32)]
        scratch += [pltpu.VMEM((CH * CP,), jnp.float32),
                    pltpu.VMEM((NN * CP,), jnp.float32)]
    scratch += [pltpu.SemaphoreType.DMA, pltpu.SemaphoreType.DMA]

    @functools.partial(
        pl.kernel, out_type=tuple(out_type), mesh=_sc_mesh(),
        scratch_types=tuple(scratch),
        compiler_params=pltpu.CompilerParams(needs_layout_passes=False))
    def k(*refs):
        if with_cd:
            (h_hbm, cp_hbm, rows_hbm, cols_hbm,
             hr_hbm, hc_hbm, cd_hbm,
             ir_v, ic_v, hr_v, hc_v, hbr_v, hbc_v, cd_v, ctab_v,
             gsem, wsem) = refs
            pltpu.sync_copy(cp_hbm, ctab_v)
        else:
            (h_hbm, rows_hbm, cols_hbm,
             hr_hbm, hc_hbm,
             ir_v, ic_v, hr_v, hc_v, hbr_v, hbc_v, gsem, wsem) = refs
        wid = lax.axis_index("s") * NC + lax.axis_index("c")
        lane = lax.iota(jnp.int32, 16)
        pltpu.sync_copy(rows_hbm.at[wid], ir_v)
        pltpu.sync_copy(cols_hbm.at[wid], ic_v)
        pltpu.async_copy(h_hbm.at[ir_v.at[0]], hr_v.at[0], gsem)
        pltpu.async_copy(h_hbm.at[ic_v.at[0]], hc_v.at[0], gsem)

        def pack_rows(p, src_v, dst_v):
            def prow(r, carry):
                for c in range(DH // 32):
                    a = src_v[p, r, pl.ds(c * 32, 16)]
                    b = src_v[p, r, pl.ds(c * 32 + 16, 16)]
                    ai = plsc.bitcast(a, jnp.int32)
                    bi = plsc.bitcast(b, jnp.int32)
                    lo = lax.shift_right_logical(ai + 0x8000, 16)
                    hi = (bi + 0x8000) & jnp.int32(-65536)
                    dst_v[pl.ds(r * DW + c * 16, 16)] = lo | hi
                return carry

            lax.fori_loop(0, CH, prow, 0)

        def body(i, carry):
            p = lax.rem(i, 2)
            pn = lax.rem(i + 1, 2)
            base = wid * EPW + i * CH
            basen = base + CH

            @pl.when(i + 1 < NCH)
            def _():
                pltpu.async_copy(h_hbm.at[ir_v.at[i + 1]], hr_v.at[pn], gsem)
                pltpu.async_copy(h_hbm.at[ic_v.at[i + 1]], hc_v.at[pn], gsem)

            if with_cd:
                for e in range(0, CH, 16):
                    rv = ir_v[i, pl.ds(e, 16)] * CP
                    cv = ic_v[i, pl.ds(e, 16)] * CP
                    row = (lane + e) * CP
                    for d in range(CP):
                        a = plsc.load_gather(ctab_v, [rv + d])
                        b = plsc.load_gather(ctab_v, [cv + d])
                        plsc.store_scatter(cd_v, [row + d], a - b)
                pltpu.sync_copy(cd_v, cd_hbm.at[pl.ds(base * CP, CH * CP)])
            pltpu.make_async_copy(h_hbm.at[ir_v.at[i]], hr_v.at[p],
                                  gsem).wait()
            pltpu.make_async_copy(h_hbm.at[ic_v.at[i]], hc_v.at[p],
                                  gsem).wait()

            @pl.when(i >= 1)
            def _():
                pltpu.make_async_copy(
                    hbr_v, hr_hbm.at[pl.ds((base - CH) * DW, CH * DW)],
                    wsem).wait()
                pltpu.make_async_copy(
                    hbc_v, hc_hbm.at[pl.ds((base - CH) * DW, CH * DW)],
                    wsem).wait()

            pack_rows(p, hr_v, hbr_v)
            pack_rows(p, hc_v, hbc_v)
            pltpu.async_copy(hbr_v, hr_hbm.at[pl.ds(base * DW, CH * DW)],
                             wsem)
            pltpu.async_copy(hbc_v, hc_hbm.at[pl.ds(base * DW, CH * DW)],
                             wsem)
            return carry

        lax.fori_loop(0, NCH, body, 0)
        lastb = (wid * EPW + (NCH - 1) * CH) * DW
        pltpu.make_async_copy(hbr_v, hr_hbm.at[pl.ds(lastb, CH * DW)],
                              wsem).wait()
        pltpu.make_async_copy(hbc_v, hc_hbm.at[pl.ds(lastb, CH * DW)],
                              wsem).wait()

    if with_cd:
        return k(h, coordp, rows3, cols3)
    return k(h, rows3, cols3)


def _sc_scatter(m, rows3, z1, cmsg=None, z2=None):
    out_type = [jax.ShapeDtypeStruct((NC, NN, DH), jnp.float32)]
    scratch = [pltpu.VMEM((NCH, CH), jnp.int32),
               pltpu.VMEM((2, CH, DH), jnp.float32),
               pltpu.VMEM_SHARED((NN, DH), jnp.float32),
               pltpu.SemaphoreType.DMA]

    @functools.partial(
        pl.kernel, out_type=tuple(out_type), mesh=_sc_mesh(),
        scratch_types=tuple(scratch),
        compiler_params=pltpu.CompilerParams(needs_layout_passes=False))
    def k(m_hbm, rows_hbm, z1_hbm, agg_hbm, idx_v, bm_v, acc_sh, rsem):
        c = lax.axis_index("c")
        s = lax.axis_index("s")
        wid = s * NC + c
        r0 = s * RPT
        pltpu.sync_copy(rows_hbm.at[wid], idx_v)

        @pl.when(s < NS - 1)
        def _():
            pltpu.sync_copy(z1_hbm.at[pl.ds(r0, RPT)], acc_sh.at[pl.ds(r0, RPT)])

        @pl.when(s == NS - 1)
        def _():
            pltpu.sync_copy(z1_hbm.at[pl.ds((NS - 1) * RPT, RPT_LAST)],
                            acc_sh.at[pl.ds((NS - 1) * RPT, RPT_LAST)])

        plsc.subcore_barrier()
        wbase = wid * EPW
        pltpu.async_copy(m_hbm.at[pl.ds(wbase, CH)], bm_v.at[0], rsem)

        def body(i, carry):
            p = lax.rem(i, 2)
            pn = lax.rem(i + 1, 2)
            base = wbase + i * CH

            @pl.when(i + 1 < NCH)
            def _():
                pltpu.async_copy(m_hbm.at[pl.ds(base + CH, CH)],
                                 bm_v.at[pn], rsem)

            pltpu.make_async_copy(m_hbm.at[pl.ds(base, CH)], bm_v.at[p],
                                  rsem).wait()
            pltpu.sync_copy(bm_v.at[p], acc_sh.at[idx_v.at[i]], add=True)
            return carry

        lax.fori_loop(0, NCH, body, 0)
        plsc.subcore_barrier()

        @pl.when(s < NS - 1)
        def _():
            pltpu.sync_copy(acc_sh.at[pl.ds(r0, RPT)],
                            agg_hbm.at[c, pl.ds(r0, RPT)])

        @pl.when(s == NS - 1)
        def _():
            pltpu.sync_copy(acc_sh.at[pl.ds((NS - 1) * RPT, RPT_LAST)],
                            agg_hbm.at[c, pl.ds((NS - 1) * RPT, RPT_LAST)])

    return k(m, rows3, z1)


def _sc_scatter_coord(cmsg, rows3, z2):
    out_type = jax.ShapeDtypeStruct((NW * NNCP_PAD,), jnp.float32)
    scratch = [pltpu.VMEM((NCH, CH), jnp.int32),
               pltpu.VMEM((CH * CP,), jnp.float32),
               pltpu.VMEM((NN * CP,), jnp.float32)]

    @functools.partial(
        pl.kernel, out_type=out_type, mesh=_sc_mesh(),
        scratch_types=tuple(scratch),
        compiler_params=pltpu.CompilerParams(needs_layout_passes=False))
    def k(cm_hbm, rows_hbm, z2_hbm, cs_hbm, idx_v, bc_v, cpriv_v):
        c = lax.axis_index("c")
        s = lax.axis_index("s")
        wid = s * NC + c
        lane = lax.iota(jnp.int32, 16)
        pltpu.sync_copy(rows_hbm.at[wid], idx_v)
        pltpu.sync_copy(z2_hbm, cpriv_v)
        wbase = wid * EPW

        def body(i, carry):
            base = wbase + i * CH
            pltpu.sync_copy(cm_hbm.at[pl.ds(base * CP, CH * CP)], bc_v)
            ones = jnp.full((16,), 1.0, jnp.float32)
            for e in range(0, CH, 16):
                rv = idx_v[i, pl.ds(e, 16)] * CP
                row = (lane + e) * CP
                for d in range(3):
                    v = plsc.load_gather(bc_v, [row + d])
                    plsc.addupdate_scatter(cpriv_v, [rv + d], v)
                plsc.addupdate_scatter(cpriv_v, [rv + 3], ones)
            return carry

        lax.fori_loop(0, NCH, body, 0)
        pltpu.sync_copy(cpriv_v, cs_hbm.at[pl.ds(wid * NNCP_PAD, NNCP)])

    return k(cmsg, rows3, z2)


def _full(arr):
    return pl.BlockSpec(arr.shape, lambda *_: tuple(0 for _ in arr.shape))


def _silu(x):
    return x * jax.nn.sigmoid(x)


def _tc_edge(hr, hc, cdm, ew, with_coord, block=2000):
    (w0a, w0b, w0r, b0, w1, b1, wat_t, bat, wc, bc, ww_t) = ew
    ne = hr.shape[0]
    grid = (ne // block,)

    def body(*refs):
        if with_coord:
            (hr_r, hc_r, cd_r, w0a_r, w0b_r, w0r_r, b0_r, w1_r, b1_r,
             wat_r, bat_r, wc_r, bc_r, ww_r, m_r, cm_r) = refs
        else:
            (hr_r, hc_r, cd_r, w0a_r, w0b_r, w0r_r, b0_r, w1_r, b1_r,
             wat_r, bat_r, m_r) = refs
        bf = jnp.bfloat16
        cd = cd_r[...]
        rad = (cd[:, 0:1] * cd[:, 0:1] + cd[:, 1:2] * cd[:, 1:2]
               + cd[:, 2:3] * cd[:, 2:3])
        e0 = (jnp.dot(hr_r[...], w0a_r[...],
                      preferred_element_type=jnp.float32)
              + jnp.dot(hc_r[...], w0b_r[...],
                        preferred_element_type=jnp.float32)
              + rad * w0r_r[...] + b0_r[...])
        e0 = _silu(e0)
        e1 = _silu(jnp.dot(e0.astype(bf), w1_r[...],
                           preferred_element_type=jnp.float32) + b1_r[...])
        e1b = e1.astype(bf)
        att = jax.nn.sigmoid(
            jnp.dot(e1b, wat_r[...],
                    preferred_element_type=jnp.float32)[:, 0:1] + bat_r[...])
        m = e1 * att
        m_r[...] = m
        if with_coord:
            cmi = _silu(jnp.dot(m.astype(bf), wc_r[...],
                                preferred_element_type=jnp.float32) + bc_r[...])
            cw = jnp.tanh(jnp.dot(cmi.astype(bf), ww_r[...],
                                  preferred_element_type=jnp.float32)[:, 0:1])
            cdn = cd / (jnp.sqrt(rad) + EPS)
            cm_r[...] = cdn * cw

    ins = [hr, hc, cdm, w0a, w0b, w0r, b0, w1, b1, wat_t, bat]
    in_specs = [pl.BlockSpec((block, DH), lambda i: (i, 0)),
                pl.BlockSpec((block, DH), lambda i: (i, 0)),
                pl.BlockSpec((block, CP), lambda i: (i, 0))] + \
               [_full(a) for a in (w0a, w0b, w0r, b0, w1, b1, wat_t, bat)]
    out_shape = [jax.ShapeDtypeStruct((ne, DH), jnp.float32)]
    out_specs = [pl.BlockSpec((block, DH), lambda i: (i, 0))]
    if with_coord:
        ins += [wc, bc, ww_t]
        in_specs += [_full(a) for a in (wc, bc, ww_t)]
        out_shape += [jax.ShapeDtypeStruct((ne, CP), jnp.float32)]
        out_specs += [pl.BlockSpec((block, CP), lambda i: (i, 0))]
    out = pl.pallas_call(body, grid=grid, in_specs=in_specs,
                         out_specs=out_specs, out_shape=out_shape)(*ins)
    return out if with_coord else (out[0],)


def _tc_reduce_nw(cs_flat, block=4096):
    x = cs_flat.reshape(NW, NNCP_PAD)
    grid = (NNCP_PAD // block,)

    def body(x_r, o_r):
        o_r[...] = jnp.sum(x_r[...], axis=0)

    out = pl.pallas_call(
        body, grid=grid,
        in_specs=[pl.BlockSpec((NW, block), lambda i: (0, i))],
        out_specs=pl.BlockSpec((block,), lambda i: (i,)),
        out_shape=jax.ShapeDtypeStruct((NNCP_PAD,), jnp.float32))(x)
    return out[:NNCP].reshape(NN, CP)


def _tc_node(h, agg_p, nw, with_coord=False, cs=None, coordp=None, block=2000,
             emit16=False):
    wh, wa, b0, w1, b1 = nw
    n = h.shape[0]
    npart = agg_p.shape[0]
    grid = (n // block,)

    def body(*refs):
        mask3 = (lax.broadcasted_iota(jnp.int32, (1, CP), 1) < 3
                 ).astype(jnp.float32)
        it = iter(refs)
        h_r = next(it)
        ag_r = next(it)
        wh_r = next(it)
        wa_r = next(it)
        b0_r = next(it)
        w1_r = next(it)
        b1_r = next(it)
        cs_r = next(it) if with_coord else None
        cp_r = next(it) if with_coord else None
        hn_r = next(it)
        h16_r = next(it) if emit16 else None
        cn_r = next(it) if with_coord else None
        agg = ag_r[0]
        for p in range(1, npart):
            agg = agg + ag_r[p]
        h = h_r[...]
        t = _silu(jnp.dot(h, wh_r[...], preferred_element_type=jnp.float32)
                  + jnp.dot(agg, wa_r[...], preferred_element_type=jnp.float32)
                  + b0_r[...])
        out = jnp.dot(t, w1_r[...], preferred_element_type=jnp.float32) + b1_r[...]
        hn = h + out
        hn_r[...] = hn
        if emit16:
            h16_r[...] = hn.astype(jnp.bfloat16)
        if with_coord:
            csv = cs_r[...]
            cnt = jnp.maximum(csv[:, 3:4], 1.0)
            cn_r[...] = cp_r[...] + (csv / cnt) * mask3

    ins = [h, agg_p, wh, wa, b0, w1, b1]
    in_specs = [pl.BlockSpec((block, DH), lambda i: (i, 0)),
                pl.BlockSpec((npart, block, DH), lambda i: (0, i, 0))] + \
               [_full(a) for a in (wh, wa, b0, w1, b1)]
    if with_coord:
        ins += [cs, coordp]
        in_specs += [pl.BlockSpec((block, CP), lambda i: (i, 0)),
                     pl.BlockSpec((block, CP), lambda i: (i, 0))]
    out_shape = [jax.ShapeDtypeStruct((n, DH), jnp.float32)]
    out_specs = [pl.BlockSpec((block, DH), lambda i: (i, 0))]
    if emit16:
        out_shape += [jax.ShapeDtypeStruct((n, DH), jnp.bfloat16)]
        out_specs += [pl.BlockSpec((block, DH), lambda i: (i, 0))]
    if with_coord:
        out_shape += [jax.ShapeDtypeStruct((n, CP), jnp.float32)]
        out_specs += [pl.BlockSpec((block, CP), lambda i: (i, 0))]
    out = pl.pallas_call(body, grid=grid, in_specs=in_specs,
                         out_specs=out_specs, out_shape=out_shape)(*ins)
    return out


def _tc_lin(x, w, b, block=None, emit16=False):
    m, kdim = x.shape
    dout = w.shape[1]
    if block is None:
        block = m if m <= 2000 else 2000
    grid = (m // block,)
    b2 = b.reshape(1, dout)

    def body(*refs):
        if emit16:
            x_r, w_r, b_r, y_r, y16_r = refs
        else:
            x_r, w_r, b_r, y_r = refs
        y = (jnp.dot(x_r[...], w_r[...], preferred_element_type=jnp.float32)
             + b_r[...])
        y_r[...] = y
        if emit16:
            y16_r[...] = y.astype(jnp.bfloat16)

    out_shape = [jax.ShapeDtypeStruct((m, dout), jnp.float32)]
    out_specs = [pl.BlockSpec((block, dout), lambda i: (i, 0))]
    if emit16:
        out_shape += [jax.ShapeDtypeStruct((m, dout), jnp.bfloat16)]
        out_specs += [pl.BlockSpec((block, dout), lambda i: (i, 0))]
    out = pl.pallas_call(
        body, grid=grid,
        in_specs=[pl.BlockSpec((block, kdim), lambda i: (i, 0)),
                  _full(w), _full(b2)],
        out_specs=out_specs,
        out_shape=out_shape)(x, w, b2)
    return out if emit16 else out[0]


def _tc_pool(x, bidx, block=2000):
    n, f = x.shape
    grid = (n // block,)
    last = n // block - 1

    def body(x_r, b_r, o_r):
        i = pl.program_id(0)
        oh = (b_r[...] == lax.broadcasted_iota(jnp.int32, (1, BB), 1)
              ).astype(jnp.float32)
        part = lax.dot_general(oh, x_r[...], (((0,), (0,)), ((), ())),
                               preferred_element_type=jnp.float32)

        @pl.when(i == 0)
        def _():
            o_r[...] = part

        @pl.when(i > 0)
        def _():
            o_r[...] = o_r[...] + part

        @pl.when(i == last)
        def _():
            s = o_r[...]
            o_r[...] = s / jnp.maximum(s[:, f - 1:f], 1.0)

    return pl.pallas_call(
        body, grid=grid,
        in_specs=[pl.BlockSpec((block, f), lambda i: (i, 0)),
                  pl.BlockSpec((block, 1), lambda i: (i, 0))],
        out_specs=pl.BlockSpec((BB, f), lambda i: (0, 0)),
        out_shape=jax.ShapeDtypeStruct((BB, f), jnp.float32))(x, bidx)


def _tc_small(h64, coords, rmat, cmat, cnt2, ew, with_coord,
              node_w=None):
    (w0a, w0b, w0r, b0, w1, b1, wat_t, bat, wc, bc, ww_t) = ew
    node_mlp = node_w is not None

    def body(*refs):
        mask3 = (lax.broadcasted_iota(jnp.int32, (1, CP), 1) < 3
                 ).astype(jnp.float32)
        it = iter(refs)
        h_r = next(it); cs_r = next(it); r_r = next(it); c_r = next(it)
        cnt_r = next(it)
        w0a_r = next(it); w0b_r = next(it); w0r_r = next(it); b0_r = next(it)
        w1_r = next(it); b1_r = next(it); wat_r = next(it); bat_r = next(it)
        if with_coord:
            wc_r = next(it); bc_r = next(it); ww_r = next(it)
        if node_mlp:
            nwh_r = next(it); nwa_r = next(it); nb0_r = next(it)
            nw1_r = next(it); nb1_r = next(it)
        o1_r = next(it)
        if with_coord:
            o2_r = next(it)
        h = h_r[...]
        cso = cs_r[...]
        rm = r_r[...]
        cm = c_r[...]
        hr = jnp.dot(rm, h, preferred_element_type=jnp.float32)
        hc = jnp.dot(cm, h, preferred_element_type=jnp.float32)
        cr = jnp.dot(rm, cso, preferred_element_type=jnp.float32)
        cc = jnp.dot(cm, cso, preferred_element_type=jnp.float32)
        cd = cr - cc
        rad = (cd[:, 0:1] * cd[:, 0:1] + cd[:, 1:2] * cd[:, 1:2]
               + cd[:, 2:3] * cd[:, 2:3])
        bfd = jnp.bfloat16
        e0 = _silu(jnp.dot(hr.astype(bfd), w0a_r[...],
                           preferred_element_type=jnp.float32)
                   + jnp.dot(hc.astype(bfd), w0b_r[...],
                             preferred_element_type=jnp.float32)
                   + rad * w0r_r[...] + b0_r[...])
        e1 = _silu(jnp.dot(e0.astype(bfd), w1_r[...],
                           preferred_element_type=jnp.float32) + b1_r[...])
        att = jax.nn.sigmoid(
            jnp.dot(e1.astype(bfd), wat_r[...],
                    preferred_element_type=jnp.float32)[:, 0:1] + bat_r[...])
        m = e1 * att
        agg = lax.dot_general(rm, m, (((0,), (0,)), ((), ())),
                              preferred_element_type=jnp.float32)
        if with_coord:
            cmi = _silu(jnp.dot(m.astype(bfd), wc_r[...],
                                preferred_element_type=jnp.float32) + bc_r[...])
            cw = jnp.tanh(jnp.dot(cmi.astype(bfd), ww_r[...],
                                  preferred_element_type=jnp.float32)[:, 0:1])
            cdn = cd / (jnp.sqrt(rad) + EPS)
            cmsg = cdn * cw
            csum = lax.dot_general(rm, cmsg, (((0,), (0,)), ((), ())),
                                   preferred_element_type=jnp.float32)
            o2_r[...] = cso + (csum / cnt_r[...]) * mask3
        if node_mlp:
            t = _silu(jnp.dot(h, nwh_r[...], preferred_element_type=jnp.float32)
                      + jnp.dot(agg, nwa_r[...], preferred_element_type=jnp.float32)
                      + nb0_r[...])
            o1_r[...] = h + (jnp.dot(t, nw1_r[...],
                                     preferred_element_type=jnp.float32)
                             + nb1_r[...])
        else:
            o1_r[...] = agg

    ins = [h64, coords, rmat, cmat, cnt2,
           w0a, w0b, w0r, b0, w1, b1, wat_t, bat]
    if with_coord:
        ins += [wc, bc, ww_t]
    if node_mlp:
        ins += list(node_w)
    in_specs = [_full(a) for a in ins]
    out_shape = [jax.ShapeDtypeStruct((BB, DH), jnp.float32)]
    out_specs = [_full(jnp.zeros((BB, DH)))]
    if with_coord:
        out_shape += [jax.ShapeDtypeStruct((BB, CP), jnp.float32)]
        out_specs += [_full(jnp.zeros((BB, CP)))]
    out = pl.pallas_call(body, grid=(1,), in_specs=in_specs,
                         out_specs=out_specs, out_shape=out_shape)(*ins)
    return out


def _tc_head(pool2, out_seq, bn1, bn2, bn3, wf_parts, bf, fbn, wfin, bfin):
    wf_a, wf_b, wf_c, wf_d = wf_parts

    def _bn(x, g, b):
        mu = jnp.mean(x, axis=0, keepdims=True)
        var = jnp.mean((x - mu) * (x - mu), axis=0, keepdims=True)
        return g * (x - mu) / jnp.sqrt(var + 1e-5) + b

    def body(p_r, os_r, g1_r, b1_r, g2_r, b2_r, g3_r, b3_r,
             wfa_r, wfb_r, wfc_r, wfd_r, bf_r, gf_r, bfg_r,
             wfin_r, bfin_r, o_r):
        p = p_r[...]
        pr = jnp.maximum(_bn(p[:, 0:128], g1_r[...], b1_r[...]), 0.0)
        pr2 = jnp.maximum(_bn(p[:, 128:192], g2_r[...], b2_r[...]), 0.0)
        pr3 = jnp.maximum(_bn(p[:, 192:224], g3_r[...], b3_r[...]), 0.0)
        ps = jnp.maximum(_bn(os_r[...], g2_r[...], b2_r[...]), 0.0)
        x = (jnp.dot(pr, wfa_r[...], preferred_element_type=jnp.float32)
             + jnp.dot(ps, wfb_r[...], preferred_element_type=jnp.float32)
             + jnp.dot(pr2, wfc_r[...], preferred_element_type=jnp.float32)
             + jnp.dot(pr3, wfd_r[...], preferred_element_type=jnp.float32)
             + bf_r[...])
        x = jnp.maximum(_bn(x, gf_r[...], bfg_r[...]), 0.0)
        x = jnp.dot(x, wfin_r[...], preferred_element_type=jnp.float32) + bfin_r[...]
        o_r[...] = jax.nn.sigmoid(x)

    ins = [pool2, out_seq,
           bn1[0].reshape(1, -1), bn1[1].reshape(1, -1),
           bn2[0].reshape(1, -1), bn2[1].reshape(1, -1),
           bn3[0].reshape(1, -1), bn3[1].reshape(1, -1),
           wf_a, wf_b, wf_c, wf_d, bf.reshape(1, -1),
           fbn[0].reshape(1, -1), fbn[1].reshape(1, -1),
           wfin, bfin.reshape(1, -1)]
    return pl.pallas_call(
        body, grid=(1,),
        in_specs=[_full(a) for a in ins],
        out_specs=_full(jnp.zeros((BB, 128))),
        out_shape=jax.ShapeDtypeStruct((BB, 128), jnp.float32))(*ins)


def _edge_weights(gp, perm=False):
    bf = jnp.bfloat16
    w0 = gp['edge0'][0]
    w0a, w0b = w0[:DH], w0[DH:2 * DH]
    if perm:
        pidx = jnp.asarray(_PERM_NP)
        w0a, w0b = w0a[pidx], w0b[pidx]
    return (w0a.astype(bf), w0b.astype(bf),
            w0[2 * DH:2 * DH + 1],
            gp['edge0'][1].reshape(1, DH),
            gp['edge1'][0].astype(bf), gp['edge1'][1].reshape(1, DH),
            jnp.pad(gp['att'][0], ((0, 0), (0, 7))).astype(bf),
            gp['att'][1].reshape(1, 1),
            gp['coord0'][0].astype(bf), gp['coord0'][1].reshape(1, DH),
            jnp.pad(gp['coordw'][0], ((0, 0), (0, 7))).astype(bf))


def _node_weights(gp):
    wn0 = gp['node0'][0]
    return (wn0[:DH], wn0[DH:], gp['node0'][1].reshape(1, DH),
            gp['node1'][0], gp['node1'][1].reshape(1, DH))


def _big_gcl(h, h16, coordp, rows, cols, gp, z1, z2, with_coord,
             cd_pre=None, emit16=False):
    ew = _edge_weights(gp, perm=True)
    nw = _node_weights(gp)
    if cd_pre is None:
        hr, hc, cd = _sc_gather(h16, coordp.reshape(-1), rows, cols,
                                with_cd=True)
        cd = cd.reshape(NE, CP)
    else:
        hr, hc = _sc_gather(h16, coordp, rows, cols, with_cd=False)
        cd = cd_pre
    hr = lax.bitcast_convert_type(hr.reshape(NE, DH // 2),
                                  jnp.bfloat16).reshape(NE, DH)
    hc = lax.bitcast_convert_type(hc.reshape(NE, DH // 2),
                                  jnp.bfloat16).reshape(NE, DH)
    e = _tc_edge(hr, hc, cd, ew, with_coord)
    if with_coord:
        m, cmsg = e
        agg_p, = _sc_scatter(m, rows, z1)
        cs_flat = _sc_scatter_coord(cmsg.reshape(-1), rows, z2)
        cs = _tc_reduce_nw(cs_flat)
        outs = _tc_node(h, agg_p, nw, True, cs, coordp, emit16=emit16)
        if emit16:
            hn, hn16, cn = outs
        else:
            (hn, cn), hn16 = outs, None
        return hn, hn16, cn, cd
    m, = e
    agg_p, = _sc_scatter(m, rows, z1)
    outs = _tc_node(h, agg_p, nw, False, emit16=emit16)
    if emit16:
        hn, hn16 = outs
    else:
        (hn,), hn16 = outs, None
    return hn, hn16, None, cd


def kernel(x_res, x_emb_seq, x_pos, edge_index, x_batch, params):
    rows = edge_index[0].astype(jnp.int32).reshape(NW, NCH, CH)
    cols = edge_index[1].astype(jnp.int32).reshape(NW, NCH, CH)
    coordp0 = jnp.pad(x_pos.astype(jnp.float32), ((0, 0), (0, CP - 3)))
    z1 = jnp.zeros((NN, DH), jnp.float32)
    z2 = jnp.zeros((NN * CP,), jnp.float32)
    bidx = x_batch.astype(jnp.int32).reshape(NN, 1)
    rmat = jnp.asarray(_R_NP)
    cmat = jnp.asarray(_C_NP)
    cnt2 = jnp.asarray(_CNT2_NP)
    ones_n = jnp.ones((NN, 1), jnp.float32)

    pooled = _tc_pool(jnp.concatenate([x_pos, ones_n], axis=1), bidx)
    pooled_cs = jnp.pad(pooled[:, :3], ((0, 0), (0, CP - 3)))

    p1 = params['egnn1']
    h = _tc_lin(x_res, p1['emb_in'][0], p1['emb_in'][1])
    h, _, c1, cd0 = _big_gcl(h, h, coordp0, rows, cols, p1['gcl'][0],
                             z1, z2, True)
    h, _, _, _ = _big_gcl(h, h, c1, rows, cols, p1['gcl'][1], z1, z2, False)
    out_res = _tc_lin(h, p1['emb_out'][0], p1['emb_out'][1])

    p2 = params['egnn2']
    h = _tc_lin(out_res, p2['emb_in'][0], p2['emb_in'][1])
    h, _, c1, _ = _big_gcl(h, h, coordp0, rows, cols, p2['gcl'][0],
                           z1, z2, True, cd_pre=cd0)
    h, _, _, _ = _big_gcl(h, h, c1, rows, cols, p2['gcl'][1], z1, z2, False)
    out_res2 = _tc_lin(h, p2['emb_out'][0], p2['emb_out'][1])

    p4 = params['egnn4']
    h4 = _tc_lin(out_res2, p4['emb_in'][0], p4['emb_in'][1])
    ew40 = _edge_weights(p4['gcl'][0])
    agg64, cs_new = _tc_small(h4[:BB], pooled_cs, rmat, cmat, cnt2, ew40, True)
    agg_full = jnp.pad(agg64, ((0, NN - BB), (0, 0)))[None]
    h4, = _tc_node(h4, agg_full, _node_weights(p4['gcl'][0]), False)
    ew41 = _edge_weights(p4['gcl'][1])
    agg64b, = _tc_small(h4[:BB], cs_new, rmat, cmat, cnt2, ew41, False)
    agg_full = jnp.pad(agg64b, ((0, NN - BB), (0, 0)))[None]
    h4, = _tc_node(h4, agg_full, _node_weights(p4['gcl'][1]), False)
    out_res3 = _tc_lin(h4, p4['emb_out'][0], p4['emb_out'][1])

    p3 = params['egnn3']
    h3 = _tc_lin(x_emb_seq, p3['emb_in'][0], p3['emb_in'][1])
    ew30 = _edge_weights(p3['gcl'][0])
    h3, cs3 = _tc_small(h3, pooled_cs, rmat, cmat, cnt2, ew30, True,
                        node_w=_node_weights(p3['gcl'][0]))
    ew31 = _edge_weights(p3['gcl'][1])
    h3, = _tc_small(h3, cs3, rmat, cmat, cnt2, ew31, False,
                    node_w=_node_weights(p3['gcl'][1]))
    out_seq = _tc_lin(h3, p3['emb_out'][0], p3['emb_out'][1])

    pool2 = _tc_pool(jnp.concatenate([out_res, out_res2, out_res3, ones_n],
                                     axis=1), bidx)

    wf = params['fc1'][0]
    wf_parts = (wf[0:128], wf[128:192], wf[192:256], wf[256:288])
    return _tc_head(pool2, out_seq, params['bn1'], params['bn2'], params['bn3'],
                    wf_parts, params['fc1'][1], params['fc1_bn'],
                    params['final'][0], params['final'][1])


# speedup vs baseline: 2.7121x; 2.7121x over previous
import functools

import numpy as np
import jax
import jax.numpy as jnp
from jax import lax
from jax.experimental import pallas as pl
from jax.experimental.pallas import tpu as pltpu
from jax.experimental.pallas import tpu_sc as plsc

NN = 10000
NE = 320000
DH = 128
CP = 4
BB = 64
EPS = 1e-8

NC, NS = 2, 16
NW = NC * NS
EPW = NE // NW
CH = 80
NCH = EPW // CH
RPT = 624
RPT_LAST = NN - (NS - 1) * RPT
NNCP = NN * CP
NNCP_PAD = 40960

@functools.lru_cache(maxsize=1)
def _sc_mesh():
    return plsc.VectorSubcoreMesh(core_axis_name="c", subcore_axis_name="s",
                                  num_cores=NC, num_subcores=NS)

_EP = 2048
_r2, _c2 = np.triu_indices(BB, k=1)
_E2 = _r2.shape[0]
_R_NP = np.zeros((_EP, BB), np.float32)
_C_NP = np.zeros((_EP, BB), np.float32)
_R_NP[np.arange(_E2), _r2] = 1.0
_C_NP[np.arange(_E2), _c2] = 1.0
_CNT2_NP = np.maximum(np.bincount(_r2, minlength=BB), 1).astype(np.float32)[:, None]

_PERM_NP = np.empty((DH,), np.int64)
for _g in range(DH // 32):
    for _i in range(16):
        _PERM_NP[_g * 32 + 2 * _i] = _g * 32 + _i
        _PERM_NP[_g * 32 + 2 * _i + 1] = _g * 32 + 16 + _i


def _sc_gather(h, coordp, rows3, cols3, with_cd=True):
    out_type = [jax.ShapeDtypeStruct((NE, DH), jnp.float32),
                jax.ShapeDtypeStruct((NE, DH), jnp.float32)]
    scratch = [pltpu.VMEM((NCH, CH), jnp.int32),
               pltpu.VMEM((NCH, CH), jnp.int32),
               pltpu.VMEM((2, CH, DH), jnp.float32),
               pltpu.VMEM((2, CH, DH), jnp.float32)]
    if with_cd:
        out_type += [jax.ShapeDtypeStruct((NE * CP,), jnp.float32)]
        scratch += [pltpu.VMEM((CH * CP,), jnp.float32),
                    pltpu.VMEM((NN * CP,), jnp.float32)]
    scratch += [pltpu.SemaphoreType.DMA, pltpu.SemaphoreType.DMA]

    @functools.partial(
        pl.kernel, out_type=tuple(out_type), mesh=_sc_mesh(),
        scratch_types=tuple(scratch),
        compiler_params=pltpu.CompilerParams(needs_layout_passes=False))
    def k(*refs):
        if with_cd:
            (h_hbm, cp_hbm, rows_hbm, cols_hbm,
             hr_hbm, hc_hbm, cd_hbm,
             ir_v, ic_v, hr_v, hc_v, cd_v, ctab_v,
             gsem, wsem) = refs
            pltpu.sync_copy(cp_hbm, ctab_v)
        else:
            (h_hbm, rows_hbm, cols_hbm,
             hr_hbm, hc_hbm,
             ir_v, ic_v, hr_v, hc_v, gsem, wsem) = refs
        wid = lax.axis_index("s") * NC + lax.axis_index("c")
        lane = lax.iota(jnp.int32, 16)
        pltpu.sync_copy(rows_hbm.at[wid], ir_v)
        pltpu.sync_copy(cols_hbm.at[wid], ic_v)
        pltpu.async_copy(h_hbm.at[ir_v.at[0]], hr_v.at[0], gsem)
        pltpu.async_copy(h_hbm.at[ic_v.at[0]], hc_v.at[0], gsem)

        def body(i, carry):
            p = lax.rem(i, 2)
            pn = lax.rem(i + 1, 2)
            base = wid * EPW + i * CH
            basen = base + CH

            @pl.when(i >= 1)
            def _():
                pltpu.make_async_copy(
                    hr_v.at[pn], hr_hbm.at[pl.ds(basen - 2 * CH, CH)],
                    wsem).wait()
                pltpu.make_async_copy(
                    hc_v.at[pn], hc_hbm.at[pl.ds(basen - 2 * CH, CH)],
                    wsem).wait()

            @pl.when(i + 1 < NCH)
            def _():
                pltpu.async_copy(h_hbm.at[ir_v.at[i + 1]], hr_v.at[pn], gsem)
                pltpu.async_copy(h_hbm.at[ic_v.at[i + 1]], hc_v.at[pn], gsem)

            if with_cd:
                for e in range(0, CH, 16):
                    rv = ir_v[i, pl.ds(e, 16)] * CP
                    cv = ic_v[i, pl.ds(e, 16)] * CP
                    row = (lane + e) * CP
                    for d in range(CP):
                        a = plsc.load_gather(ctab_v, [rv + d])
                        b = plsc.load_gather(ctab_v, [cv + d])
                        plsc.store_scatter(cd_v, [row + d], a - b)
                pltpu.sync_copy(cd_v, cd_hbm.at[pl.ds(base * CP, CH * CP)])
            pltpu.make_async_copy(h_hbm.at[ir_v.at[i]], hr_v.at[p],
                                  gsem).wait()
            pltpu.make_async_copy(h_hbm.at[ic_v.at[i]], hc_v.at[p],
                                  gsem).wait()
            pltpu.async_copy(hr_v.at[p], hr_hbm.at[pl.ds(base, CH)], wsem)
            pltpu.async_copy(hc_v.at[p], hc_hbm.at[pl.ds(base, CH)], wsem)
            return carry

        lax.fori_loop(0, NCH, body, 0)
        pf = (NCH - 1) % 2
        lastb = wid * EPW + (NCH - 1) * CH
        pltpu.make_async_copy(hr_v.at[pf], hr_hbm.at[pl.ds(lastb, CH)],
                              wsem).wait()
        pltpu.make_async_copy(hc_v.at[pf], hc_hbm.at[pl.ds(lastb, CH)],
                              wsem).wait()

    if with_cd:
        return k(h, coordp, rows3, cols3)
    return k(h, rows3, cols3)


def _sc_scatter(m, rows3, z1, cmsg=None, z2=None):
    out_type = [jax.ShapeDtypeStruct((NC, NN, DH), jnp.float32)]
    scratch = [pltpu.VMEM((NCH, CH), jnp.int32),
               pltpu.VMEM((2, CH, DH), jnp.float32),
               pltpu.VMEM_SHARED((NN, DH), jnp.float32),
               pltpu.SemaphoreType.DMA]

    @functools.partial(
        pl.kernel, out_type=tuple(out_type), mesh=_sc_mesh(),
        scratch_types=tuple(scratch),
        compiler_params=pltpu.CompilerParams(needs_layout_passes=False))
    def k(m_hbm, rows_hbm, z1_hbm, agg_hbm, idx_v, bm_v, acc_sh, rsem):
        c = lax.axis_index("c")
        s = lax.axis_index("s")
        wid = s * NC + c
        r0 = s * RPT
        pltpu.sync_copy(rows_hbm.at[wid], idx_v)

        @pl.when(s < NS - 1)
        def _():
            pltpu.sync_copy(z1_hbm.at[pl.ds(r0, RPT)], acc_sh.at[pl.ds(r0, RPT)])

        @pl.when(s == NS - 1)
        def _():
            pltpu.sync_copy(z1_hbm.at[pl.ds((NS - 1) * RPT, RPT_LAST)],
                            acc_sh.at[pl.ds((NS - 1) * RPT, RPT_LAST)])

        plsc.subcore_barrier()
        wbase = wid * EPW
        pltpu.async_copy(m_hbm.at[pl.ds(wbase, CH)], bm_v.at[0], rsem)

        def body(i, carry):
            p = lax.rem(i, 2)
            pn = lax.rem(i + 1, 2)
            base = wbase + i * CH

            @pl.when(i + 1 < NCH)
            def _():
                pltpu.async_copy(m_hbm.at[pl.ds(base + CH, CH)],
                                 bm_v.at[pn], rsem)

            pltpu.make_async_copy(m_hbm.at[pl.ds(base, CH)], bm_v.at[p],
                                  rsem).wait()
            pltpu.sync_copy(bm_v.at[p], acc_sh.at[idx_v.at[i]], add=True)
            return carry

        lax.fori_loop(0, NCH, body, 0)
        plsc.subcore_barrier()

        @pl.when(s < NS - 1)
        def _():
            pltpu.sync_copy(acc_sh.at[pl.ds(r0, RPT)],
                            agg_hbm.at[c, pl.ds(r0, RPT)])

        @pl.when(s == NS - 1)
        def _():
            pltpu.sync_copy(acc_sh.at[pl.ds((NS - 1) * RPT, RPT_LAST)],
                            agg_hbm.at[c, pl.ds((NS - 1) * RPT, RPT_LAST)])

    return k(m, rows3, z1)


def _sc_scatter_coord(cmsg, rows3, z2):
    out_type = jax.ShapeDtypeStruct((NW * NNCP_PAD,), jnp.float32)
    scratch = [pltpu.VMEM((NCH, CH), jnp.int32),
               pltpu.VMEM((CH * CP,), jnp.float32),
               pltpu.VMEM((NN * CP,), jnp.float32)]

    @functools.partial(
        pl.kernel, out_type=out_type, mesh=_sc_mesh(),
        scratch_types=tuple(scratch),
        compiler_params=pltpu.CompilerParams(needs_layout_passes=False))
    def k(cm_hbm, rows_hbm, z2_hbm, cs_hbm, idx_v, bc_v, cpriv_v):
        c = lax.axis_index("c")
        s = lax.axis_index("s")
        wid = s * NC + c
        lane = lax.iota(jnp.int32, 16)
        pltpu.sync_copy(rows_hbm.at[wid], idx_v)
        pltpu.sync_copy(z2_hbm, cpriv_v)
        wbase = wid * EPW

        def body(i, carry):
            base = wbase + i * CH
            pltpu.sync_copy(cm_hbm.at[pl.ds(base * CP, CH * CP)], bc_v)
            ones = jnp.full((16,), 1.0, jnp.float32)
            for e in range(0, CH, 16):
                rv = idx_v[i, pl.ds(e, 16)] * CP
                row = (lane + e) * CP
                for d in range(3):
                    v = plsc.load_gather(bc_v, [row + d])
                    plsc.addupdate_scatter(cpriv_v, [rv + d], v)
                plsc.addupdate_scatter(cpriv_v, [rv + 3], ones)
            return carry

        lax.fori_loop(0, NCH, body, 0)
        pltpu.sync_copy(cpriv_v, cs_hbm.at[pl.ds(wid * NNCP_PAD, NNCP)])

    return k(cmsg, rows3, z2)


def _full(arr):
    return pl.BlockSpec(arr.shape, lambda *_: tuple(0 for _ in arr.shape))


def _silu(x):
    return x * jax.nn.sigmoid(x)


def _tc_edge(hr, hc, cdm, ew, with_coord, block=2000):
    (w0a, w0b, w0r, b0, w1, b1, wat_t, bat, wc, bc, ww_t) = ew
    ne = hr.shape[0]
    grid = (ne // block,)

    def body(*refs):
        if with_coord:
            (hr_r, hc_r, cd_r, w0a_r, w0b_r, w0r_r, b0_r, w1_r, b1_r,
             wat_r, bat_r, wc_r, bc_r, ww_r, m_r, cm_r) = refs
        else:
            (hr_r, hc_r, cd_r, w0a_r, w0b_r, w0r_r, b0_r, w1_r, b1_r,
             wat_r, bat_r, m_r) = refs
        bf = jnp.bfloat16
        cd = cd_r[...]
        rad = (cd[:, 0:1] * cd[:, 0:1] + cd[:, 1:2] * cd[:, 1:2]
               + cd[:, 2:3] * cd[:, 2:3])
        e0 = (jnp.dot(hr_r[...].astype(bf), w0a_r[...],
                      preferred_element_type=jnp.float32)
              + jnp.dot(hc_r[...].astype(bf), w0b_r[...],
                        preferred_element_type=jnp.float32)
              + rad * w0r_r[...] + b0_r[...])
        e0 = _silu(e0)
        e1 = _silu(jnp.dot(e0.astype(bf), w1_r[...],
                           preferred_element_type=jnp.float32) + b1_r[...])
        e1b = e1.astype(bf)
        att = jax.nn.sigmoid(
            jnp.dot(e1b, wat_r[...],
                    preferred_element_type=jnp.float32)[:, 0:1] + bat_r[...])
        m = e1 * att
        m_r[...] = m
        if with_coord:
            cmi = _silu(jnp.dot(m.astype(bf), wc_r[...],
                                preferred_element_type=jnp.float32) + bc_r[...])
            cw = jnp.tanh(jnp.dot(cmi.astype(bf), ww_r[...],
                                  preferred_element_type=jnp.float32)[:, 0:1])
            cdn = cd / (jnp.sqrt(rad) + EPS)
            cm_r[...] = cdn * cw

    ins = [hr, hc, cdm, w0a, w0b, w0r, b0, w1, b1, wat_t, bat]
    in_specs = [pl.BlockSpec((block, DH), lambda i: (i, 0)),
                pl.BlockSpec((block, DH), lambda i: (i, 0)),
                pl.BlockSpec((block, CP), lambda i: (i, 0))] + \
               [_full(a) for a in (w0a, w0b, w0r, b0, w1, b1, wat_t, bat)]
    out_shape = [jax.ShapeDtypeStruct((ne, DH), jnp.float32)]
    out_specs = [pl.BlockSpec((block, DH), lambda i: (i, 0))]
    if with_coord:
        ins += [wc, bc, ww_t]
        in_specs += [_full(a) for a in (wc, bc, ww_t)]
        out_shape += [jax.ShapeDtypeStruct((ne, CP), jnp.float32)]
        out_specs += [pl.BlockSpec((block, CP), lambda i: (i, 0))]
    out = pl.pallas_call(body, grid=grid, in_specs=in_specs,
                         out_specs=out_specs, out_shape=out_shape)(*ins)
    return out if with_coord else (out[0],)


def _tc_reduce_nw(cs_flat, block=4096):
    x = cs_flat.reshape(NW, NNCP_PAD)
    grid = (NNCP_PAD // block,)

    def body(x_r, o_r):
        o_r[...] = jnp.sum(x_r[...], axis=0)

    out = pl.pallas_call(
        body, grid=grid,
        in_specs=[pl.BlockSpec((NW, block), lambda i: (0, i))],
        out_specs=pl.BlockSpec((block,), lambda i: (i,)),
        out_shape=jax.ShapeDtypeStruct((NNCP_PAD,), jnp.float32))(x)
    return out[:NNCP].reshape(NN, CP)


def _tc_node(h, agg_p, nw, with_coord=False, cs=None, coordp=None, block=2000,
             emit16=False):
    wh, wa, b0, w1, b1 = nw
    n = h.shape[0]
    npart = agg_p.shape[0]
    grid = (n // block,)

    def body(*refs):
        mask3 = (lax.broadcasted_iota(jnp.int32, (1, CP), 1) < 3
                 ).astype(jnp.float32)
        it = iter(refs)
        h_r = next(it)
        ag_r = next(it)
        wh_r = next(it)
        wa_r = next(it)
        b0_r = next(it)
        w1_r = next(it)
        b1_r = next(it)
        cs_r = next(it) if with_coord else None
        cp_r = next(it) if with_coord else None
        hn_r = next(it)
        h16_r = next(it) if emit16 else None
        cn_r = next(it) if with_coord else None
        agg = ag_r[0]
        for p in range(1, npart):
            agg = agg + ag_r[p]
        h = h_r[...]
        t = _silu(jnp.dot(h, wh_r[...], preferred_element_type=jnp.float32)
                  + jnp.dot(agg, wa_r[...], preferred_element_type=jnp.float32)
                  + b0_r[...])
        out = jnp.dot(t, w1_r[...], preferred_element_type=jnp.float32) + b1_r[...]
        hn = h + out
        hn_r[...] = hn
        if emit16:
            h16_r[...] = hn.astype(jnp.bfloat16)
        if with_coord:
            csv = cs_r[...]
            cnt = jnp.maximum(csv[:, 3:4], 1.0)
            cn_r[...] = cp_r[...] + (csv / cnt) * mask3

    ins = [h, agg_p, wh, wa, b0, w1, b1]
    in_specs = [pl.BlockSpec((block, DH), lambda i: (i, 0)),
                pl.BlockSpec((npart, block, DH), lambda i: (0, i, 0))] + \
               [_full(a) for a in (wh, wa, b0, w1, b1)]
    if with_coord:
        ins += [cs, coordp]
        in_specs += [pl.BlockSpec((block, CP), lambda i: (i, 0)),
                     pl.BlockSpec((block, CP), lambda i: (i, 0))]
    out_shape = [jax.ShapeDtypeStruct((n, DH), jnp.float32)]
    out_specs = [pl.BlockSpec((block, DH), lambda i: (i, 0))]
    if emit16:
        out_shape += [jax.ShapeDtypeStruct((n, DH), jnp.bfloat16)]
        out_specs += [pl.BlockSpec((block, DH), lambda i: (i, 0))]
    if with_coord:
        out_shape += [jax.ShapeDtypeStruct((n, CP), jnp.float32)]
        out_specs += [pl.BlockSpec((block, CP), lambda i: (i, 0))]
    out = pl.pallas_call(body, grid=grid, in_specs=in_specs,
                         out_specs=out_specs, out_shape=out_shape)(*ins)
    return out


def _tc_lin(x, w, b, block=None, emit16=False):
    m, kdim = x.shape
    dout = w.shape[1]
    if block is None:
        block = m if m <= 2000 else 2000
    grid = (m // block,)
    b2 = b.reshape(1, dout)

    def body(*refs):
        if emit16:
            x_r, w_r, b_r, y_r, y16_r = refs
        else:
            x_r, w_r, b_r, y_r = refs
        y = (jnp.dot(x_r[...], w_r[...], preferred_element_type=jnp.float32)
             + b_r[...])
        y_r[...] = y
        if emit16:
            y16_r[...] = y.astype(jnp.bfloat16)

    out_shape = [jax.ShapeDtypeStruct((m, dout), jnp.float32)]
    out_specs = [pl.BlockSpec((block, dout), lambda i: (i, 0))]
    if emit16:
        out_shape += [jax.ShapeDtypeStruct((m, dout), jnp.bfloat16)]
        out_specs += [pl.BlockSpec((block, dout), lambda i: (i, 0))]
    out = pl.pallas_call(
        body, grid=grid,
        in_specs=[pl.BlockSpec((block, kdim), lambda i: (i, 0)),
                  _full(w), _full(b2)],
        out_specs=out_specs,
        out_shape=out_shape)(x, w, b2)
    return out if emit16 else out[0]


def _tc_pool(x, bidx, block=2000):
    n, f = x.shape
    grid = (n // block,)
    last = n // block - 1

    def body(x_r, b_r, o_r):
        i = pl.program_id(0)
        oh = (b_r[...] == lax.broadcasted_iota(jnp.int32, (1, BB), 1)
              ).astype(jnp.float32)
        part = lax.dot_general(oh, x_r[...], (((0,), (0,)), ((), ())),
                               preferred_element_type=jnp.float32)

        @pl.when(i == 0)
        def _():
            o_r[...] = part

        @pl.when(i > 0)
        def _():
            o_r[...] = o_r[...] + part

        @pl.when(i == last)
        def _():
            s = o_r[...]
            o_r[...] = s / jnp.maximum(s[:, f - 1:f], 1.0)

    return pl.pallas_call(
        body, grid=grid,
        in_specs=[pl.BlockSpec((block, f), lambda i: (i, 0)),
                  pl.BlockSpec((block, 1), lambda i: (i, 0))],
        out_specs=pl.BlockSpec((BB, f), lambda i: (0, 0)),
        out_shape=jax.ShapeDtypeStruct((BB, f), jnp.float32))(x, bidx)


def _tc_small(h64, coords, rmat, cmat, cnt2, ew, with_coord,
              node_w=None):
    (w0a, w0b, w0r, b0, w1, b1, wat_t, bat, wc, bc, ww_t) = ew
    node_mlp = node_w is not None

    def body(*refs):
        mask3 = (lax.broadcasted_iota(jnp.int32, (1, CP), 1) < 3
                 ).astype(jnp.float32)
        it = iter(refs)
        h_r = next(it); cs_r = next(it); r_r = next(it); c_r = next(it)
        cnt_r = next(it)
        w0a_r = next(it); w0b_r = next(it); w0r_r = next(it); b0_r = next(it)
        w1_r = next(it); b1_r = next(it); wat_r = next(it); bat_r = next(it)
        if with_coord:
            wc_r = next(it); bc_r = next(it); ww_r = next(it)
        if node_mlp:
            nwh_r = next(it); nwa_r = next(it); nb0_r = next(it)
            nw1_r = next(it); nb1_r = next(it)
        o1_r = next(it)
        if with_coord:
            o2_r = next(it)
        h = h_r[...]
        cso = cs_r[...]
        rm = r_r[...]
        cm = c_r[...]
        hr = jnp.dot(rm, h, preferred_element_type=jnp.float32)
        hc = jnp.dot(cm, h, preferred_element_type=jnp.float32)
        cr = jnp.dot(rm, cso, preferred_element_type=jnp.float32)
        cc = jnp.dot(cm, cso, preferred_element_type=jnp.float32)
        cd = cr - cc
        rad = (cd[:, 0:1] * cd[:, 0:1] + cd[:, 1:2] * cd[:, 1:2]
               + cd[:, 2:3] * cd[:, 2:3])
        bfd = jnp.bfloat16
        e0 = _silu(jnp.dot(hr.astype(bfd), w0a_r[...],
                           preferred_element_type=jnp.float32)
                   + jnp.dot(hc.astype(bfd), w0b_r[...],
                             preferred_element_type=jnp.float32)
                   + rad * w0r_r[...] + b0_r[...])
        e1 = _silu(jnp.dot(e0.astype(bfd), w1_r[...],
                           preferred_element_type=jnp.float32) + b1_r[...])
        att = jax.nn.sigmoid(
            jnp.dot(e1.astype(bfd), wat_r[...],
                    preferred_element_type=jnp.float32)[:, 0:1] + bat_r[...])
        m = e1 * att
        agg = lax.dot_general(rm, m, (((0,), (0,)), ((), ())),
                              preferred_element_type=jnp.float32)
        if with_coord:
            cmi = _silu(jnp.dot(m.astype(bfd), wc_r[...],
                                preferred_element_type=jnp.float32) + bc_r[...])
            cw = jnp.tanh(jnp.dot(cmi.astype(bfd), ww_r[...],
                                  preferred_element_type=jnp.float32)[:, 0:1])
            cdn = cd / (jnp.sqrt(rad) + EPS)
            cmsg = cdn * cw
            csum = lax.dot_general(rm, cmsg, (((0,), (0,)), ((), ())),
                                   preferred_element_type=jnp.float32)
            o2_r[...] = cso + (csum / cnt_r[...]) * mask3
        if node_mlp:
            t = _silu(jnp.dot(h, nwh_r[...], preferred_element_type=jnp.float32)
                      + jnp.dot(agg, nwa_r[...], preferred_element_type=jnp.float32)
                      + nb0_r[...])
            o1_r[...] = h + (jnp.dot(t, nw1_r[...],
                                     preferred_element_type=jnp.float32)
                             + nb1_r[...])
        else:
            o1_r[...] = agg

    ins = [h64, coords, rmat, cmat, cnt2,
           w0a, w0b, w0r, b0, w1, b1, wat_t, bat]
    if with_coord:
        ins += [wc, bc, ww_t]
    if node_mlp:
        ins += list(node_w)
    in_specs = [_full(a) for a in ins]
    out_shape = [jax.ShapeDtypeStruct((BB, DH), jnp.float32)]
    out_specs = [_full(jnp.zeros((BB, DH)))]
    if with_coord:
        out_shape += [jax.ShapeDtypeStruct((BB, CP), jnp.float32)]
        out_specs += [_full(jnp.zeros((BB, CP)))]
    out = pl.pallas_call(body, grid=(1,), in_specs=in_specs,
                         out_specs=out_specs, out_shape=out_shape)(*ins)
    return out


def _tc_head(pool2, out_seq, bn1, bn2, bn3, wf_parts, bf, fbn, wfin, bfin):
    wf_a, wf_b, wf_c, wf_d = wf_parts

    def _bn(x, g, b):
        mu = jnp.mean(x, axis=0, keepdims=True)
        var = jnp.mean((x - mu) * (x - mu), axis=0, keepdims=True)
        return g * (x - mu) / jnp.sqrt(var + 1e-5) + b

    def body(p_r, os_r, g1_r, b1_r, g2_r, b2_r, g3_r, b3_r,
             wfa_r, wfb_r, wfc_r, wfd_r, bf_r, gf_r, bfg_r,
             wfin_r, bfin_r, o_r):
        p = p_r[...]
        pr = jnp.maximum(_bn(p[:, 0:128], g1_r[...], b1_r[...]), 0.0)
        pr2 = jnp.maximum(_bn(p[:, 128:192], g2_r[...], b2_r[...]), 0.0)
        pr3 = jnp.maximum(_bn(p[:, 192:224], g3_r[...], b3_r[...]), 0.0)
        ps = jnp.maximum(_bn(os_r[...], g2_r[...], b2_r[...]), 0.0)
        x = (jnp.dot(pr, wfa_r[...], preferred_element_type=jnp.float32)
             + jnp.dot(ps, wfb_r[...], preferred_element_type=jnp.float32)
             + jnp.dot(pr2, wfc_r[...], preferred_element_type=jnp.float32)
             + jnp.dot(pr3, wfd_r[...], preferred_element_type=jnp.float32)
             + bf_r[...])
        x = jnp.maximum(_bn(x, gf_r[...], bfg_r[...]), 0.0)
        x = jnp.dot(x, wfin_r[...], preferred_element_type=jnp.float32) + bfin_r[...]
        o_r[...] = jax.nn.sigmoid(x)

    ins = [pool2, out_seq,
           bn1[0].reshape(1, -1), bn1[1].reshape(1, -1),
           bn2[0].reshape(1, -1), bn2[1].reshape(1, -1),
           bn3[0].reshape(1, -1), bn3[1].reshape(1, -1),
           wf_a, wf_b, wf_c, wf_d, bf.reshape(1, -1),
           fbn[0].reshape(1, -1), fbn[1].reshape(1, -1),
           wfin, bfin.reshape(1, -1)]
    return pl.pallas_call(
        body, grid=(1,),
        in_specs=[_full(a) for a in ins],
        out_specs=_full(jnp.zeros((BB, 128))),
        out_shape=jax.ShapeDtypeStruct((BB, 128), jnp.float32))(*ins)


def _edge_weights(gp, perm=False):
    bf = jnp.bfloat16
    w0 = gp['edge0'][0]
    w0a, w0b = w0[:DH], w0[DH:2 * DH]
    if perm:
        pidx = jnp.asarray(_PERM_NP)
        w0a, w0b = w0a[pidx], w0b[pidx]
    return (w0a.astype(bf), w0b.astype(bf),
            w0[2 * DH:2 * DH + 1],
            gp['edge0'][1].reshape(1, DH),
            gp['edge1'][0].astype(bf), gp['edge1'][1].reshape(1, DH),
            jnp.pad(gp['att'][0], ((0, 0), (0, 7))).astype(bf),
            gp['att'][1].reshape(1, 1),
            gp['coord0'][0].astype(bf), gp['coord0'][1].reshape(1, DH),
            jnp.pad(gp['coordw'][0], ((0, 0), (0, 7))).astype(bf))


def _node_weights(gp):
    wn0 = gp['node0'][0]
    return (wn0[:DH], wn0[DH:], gp['node0'][1].reshape(1, DH),
            gp['node1'][0], gp['node1'][1].reshape(1, DH))


def _big_gcl(h, h16, coordp, rows, cols, gp, z1, z2, with_coord,
             cd_pre=None, emit16=False):
    ew = _edge_weights(gp)
    nw = _node_weights(gp)
    if cd_pre is None:
        hr, hc, cd = _sc_gather(h16, coordp.reshape(-1), rows, cols,
                                with_cd=True)
        cd = cd.reshape(NE, CP)
    else:
        hr, hc = _sc_gather(h16, coordp, rows, cols, with_cd=False)
        cd = cd_pre
    e = _tc_edge(hr, hc, cd, ew, with_coord)
    if with_coord:
        m, cmsg = e
        agg_p, = _sc_scatter(m, rows, z1)
        cs_flat = _sc_scatter_coord(cmsg.reshape(-1), rows, z2)
        cs = _tc_reduce_nw(cs_flat)
        outs = _tc_node(h, agg_p, nw, True, cs, coordp, emit16=emit16)
        if emit16:
            hn, hn16, cn = outs
        else:
            (hn, cn), hn16 = outs, None
        return hn, hn16, cn, cd
    m, = e
    agg_p, = _sc_scatter(m, rows, z1)
    outs = _tc_node(h, agg_p, nw, False, emit16=emit16)
    if emit16:
        hn, hn16 = outs
    else:
        (hn,), hn16 = outs, None
    return hn, hn16, None, cd


def kernel(x_res, x_emb_seq, x_pos, edge_index, x_batch, params):
    rows = edge_index[0].astype(jnp.int32).reshape(NW, NCH, CH)
    cols = edge_index[1].astype(jnp.int32).reshape(NW, NCH, CH)
    coordp0 = jnp.pad(x_pos.astype(jnp.float32), ((0, 0), (0, CP - 3)))
    z1 = jnp.zeros((NN, DH), jnp.float32)
    z2 = jnp.zeros((NN * CP,), jnp.float32)
    bidx = x_batch.astype(jnp.int32).reshape(NN, 1)
    rmat = jnp.asarray(_R_NP)
    cmat = jnp.asarray(_C_NP)
    cnt2 = jnp.asarray(_CNT2_NP)
    ones_n = jnp.ones((NN, 1), jnp.float32)

    pooled = _tc_pool(jnp.concatenate([x_pos, ones_n], axis=1), bidx)
    pooled_cs = jnp.pad(pooled[:, :3], ((0, 0), (0, CP - 3)))

    p1 = params['egnn1']
    h = _tc_lin(x_res, p1['emb_in'][0], p1['emb_in'][1])
    h, _, c1, cd0 = _big_gcl(h, h, coordp0, rows, cols, p1['gcl'][0],
                             z1, z2, True)
    h, _, _, _ = _big_gcl(h, h, c1, rows, cols, p1['gcl'][1], z1, z2, False)
    out_res = _tc_lin(h, p1['emb_out'][0], p1['emb_out'][1])

    p2 = params['egnn2']
    h = _tc_lin(out_res, p2['emb_in'][0], p2['emb_in'][1])
    h, _, c1, _ = _big_gcl(h, h, coordp0, rows, cols, p2['gcl'][0],
                           z1, z2, True, cd_pre=cd0)
    h, _, _, _ = _big_gcl(h, h, c1, rows, cols, p2['gcl'][1], z1, z2, False)
    out_res2 = _tc_lin(h, p2['emb_out'][0], p2['emb_out'][1])

    p4 = params['egnn4']
    h4 = _tc_lin(out_res2, p4['emb_in'][0], p4['emb_in'][1])
    ew40 = _edge_weights(p4['gcl'][0])
    agg64, cs_new = _tc_small(h4[:BB], pooled_cs, rmat, cmat, cnt2, ew40, True)
    agg_full = jnp.pad(agg64, ((0, NN - BB), (0, 0)))[None]
    h4, = _tc_node(h4, agg_full, _node_weights(p4['gcl'][0]), False)
    ew41 = _edge_weights(p4['gcl'][1])
    agg64b, = _tc_small(h4[:BB], cs_new, rmat, cmat, cnt2, ew41, False)
    agg_full = jnp.pad(agg64b, ((0, NN - BB), (0, 0)))[None]
    h4, = _tc_node(h4, agg_full, _node_weights(p4['gcl'][1]), False)
    out_res3 = _tc_lin(h4, p4['emb_out'][0], p4['emb_out'][1])

    p3 = params['egnn3']
    h3 = _tc_lin(x_emb_seq, p3['emb_in'][0], p3['emb_in'][1])
    ew30 = _edge_weights(p3['gcl'][0])
    h3, cs3 = _tc_small(h3, pooled_cs, rmat, cmat, cnt2, ew30, True,
                        node_w=_node_weights(p3['gcl'][0]))
    ew31 = _edge_weights(p3['gcl'][1])
    h3, = _tc_small(h3, cs3, rmat, cmat, cnt2, ew31, False,
                    node_w=_node_weights(p3['gcl'][1]))
    out_seq = _tc_lin(h3, p3['emb_out'][0], p3['emb_out'][1])

    pool2 = _tc_pool(jnp.concatenate([out_res, out_res2, out_res3, ones_n],
                                     axis=1), bidx)

    wf = params['fc1'][0]
    wf_parts = (wf[0:128], wf[128:192], wf[192:256], wf[256:288])
    return _tc_head(pool2, out_seq, params['bn1'], params['bn2'], params['bn3'],
                    wf_parts, params['fc1'][1], params['fc1_bn'],
                    params['final'][0], params['final'][1])


# speedup vs baseline: 2.7474x; 1.0130x over previous
import functools

import numpy as np
import jax
import jax.numpy as jnp
from jax import lax
from jax.experimental import pallas as pl
from jax.experimental.pallas import tpu as pltpu
from jax.experimental.pallas import tpu_sc as plsc

NN = 10000
NE = 320000
DH = 128
CP = 4
BB = 64
EPS = 1e-8

NC, NS = 2, 16
NW = NC * NS
EPW = NE // NW
CH = 80
NCH = EPW // CH
RPT = 624
RPT_LAST = NN - (NS - 1) * RPT
NNCP = NN * CP
NNCP_PAD = 40960

@functools.lru_cache(maxsize=1)
def _sc_mesh():
    return plsc.VectorSubcoreMesh(core_axis_name="c", subcore_axis_name="s",
                                  num_cores=NC, num_subcores=NS)

_EP = 2048
_r2, _c2 = np.triu_indices(BB, k=1)
_E2 = _r2.shape[0]
_R_NP = np.zeros((_EP, BB), np.float32)
_C_NP = np.zeros((_EP, BB), np.float32)
_R_NP[np.arange(_E2), _r2] = 1.0
_C_NP[np.arange(_E2), _c2] = 1.0
_CNT2_NP = np.maximum(np.bincount(_r2, minlength=BB), 1).astype(np.float32)[:, None]

_PERM_NP = np.empty((DH,), np.int64)
for _g in range(DH // 32):
    for _i in range(16):
        _PERM_NP[_g * 32 + 2 * _i] = _g * 32 + _i
        _PERM_NP[_g * 32 + 2 * _i + 1] = _g * 32 + 16 + _i


def _sc_gather(h, coordp, rows3, cols3, with_cd=True):
    out_type = [jax.ShapeDtypeStruct((NE, DH), jnp.float32),
                jax.ShapeDtypeStruct((NE, DH), jnp.float32)]
    scratch = [pltpu.VMEM((NCH, CH), jnp.int32),
               pltpu.VMEM((NCH, CH), jnp.int32),
               pltpu.VMEM((2, CH, DH), jnp.float32),
               pltpu.VMEM((2, CH, DH), jnp.float32)]
    if with_cd:
        out_type += [jax.ShapeDtypeStruct((NE * CP,), jnp.float32)]
        scratch += [pltpu.VMEM((CH * CP,), jnp.float32),
                    pltpu.VMEM((NN * CP,), jnp.float32)]
    scratch += [pltpu.SemaphoreType.DMA, pltpu.SemaphoreType.DMA]

    @functools.partial(
        pl.kernel, out_type=tuple(out_type), mesh=_sc_mesh(),
        scratch_types=tuple(scratch),
        compiler_params=pltpu.CompilerParams(needs_layout_passes=False))
    def k(*refs):
        if with_cd:
            (h_hbm, cp_hbm, rows_hbm, cols_hbm,
             hr_hbm, hc_hbm, cd_hbm,
             ir_v, ic_v, hr_v, hc_v, cd_v, ctab_v,
             gsem, wsem) = refs
            pltpu.sync_copy(cp_hbm, ctab_v)
        else:
            (h_hbm, rows_hbm, cols_hbm,
             hr_hbm, hc_hbm,
             ir_v, ic_v, hr_v, hc_v, gsem, wsem) = refs
        wid = lax.axis_index("s") * NC + lax.axis_index("c")
        lane = lax.iota(jnp.int32, 16)
        pltpu.sync_copy(rows_hbm.at[wid], ir_v)
        pltpu.sync_copy(cols_hbm.at[wid], ic_v)
        pltpu.async_copy(h_hbm.at[ir_v.at[0]], hr_v.at[0], gsem)
        pltpu.async_copy(h_hbm.at[ic_v.at[0]], hc_v.at[0], gsem)

        def body(i, carry):
            p = lax.rem(i, 2)
            pn = lax.rem(i + 1, 2)
            base = wid * EPW + i * CH
            basen = base + CH

            @pl.when(i >= 1)
            def _():
                pltpu.make_async_copy(
                    hr_v.at[pn], hr_hbm.at[pl.ds(basen - 2 * CH, CH)],
                    wsem).wait()
                pltpu.make_async_copy(
                    hc_v.at[pn], hc_hbm.at[pl.ds(basen - 2 * CH, CH)],
                    wsem).wait()

            @pl.when(i + 1 < NCH)
            def _():
                pltpu.async_copy(h_hbm.at[ir_v.at[i + 1]], hr_v.at[pn], gsem)
                pltpu.async_copy(h_hbm.at[ic_v.at[i + 1]], hc_v.at[pn], gsem)

            if with_cd:
                for e in range(0, CH, 16):
                    rv = ir_v[i, pl.ds(e, 16)] * CP
                    cv = ic_v[i, pl.ds(e, 16)] * CP
                    row = (lane + e) * CP
                    for d in range(CP):
                        a = plsc.load_gather(ctab_v, [rv + d])
                        b = plsc.load_gather(ctab_v, [cv + d])
                        plsc.store_scatter(cd_v, [row + d], a - b)
                pltpu.sync_copy(cd_v, cd_hbm.at[pl.ds(base * CP, CH * CP)])
            pltpu.make_async_copy(h_hbm.at[ir_v.at[i]], hr_v.at[p],
                                  gsem).wait()
            pltpu.make_async_copy(h_hbm.at[ic_v.at[i]], hc_v.at[p],
                                  gsem).wait()
            pltpu.async_copy(hr_v.at[p], hr_hbm.at[pl.ds(base, CH)], wsem)
            pltpu.async_copy(hc_v.at[p], hc_hbm.at[pl.ds(base, CH)], wsem)
            return carry

        lax.fori_loop(0, NCH, body, 0)
        pf = (NCH - 1) % 2
        lastb = wid * EPW + (NCH - 1) * CH
        pltpu.make_async_copy(hr_v.at[pf], hr_hbm.at[pl.ds(lastb, CH)],
                              wsem).wait()
        pltpu.make_async_copy(hc_v.at[pf], hc_hbm.at[pl.ds(lastb, CH)],
                              wsem).wait()

    if with_cd:
        return k(h, coordp, rows3, cols3)
    return k(h, rows3, cols3)


def _sc_scatter(m, rows3, z1, cmsg=None, z2=None):
    out_type = [jax.ShapeDtypeStruct((NC, NN, DH), jnp.float32)]
    scratch = [pltpu.VMEM((NCH, CH), jnp.int32),
               pltpu.VMEM((2, CH, DH), jnp.float32),
               pltpu.VMEM_SHARED((NN, DH), jnp.float32),
               pltpu.SemaphoreType.DMA]

    @functools.partial(
        pl.kernel, out_type=tuple(out_type), mesh=_sc_mesh(),
        scratch_types=tuple(scratch),
        compiler_params=pltpu.CompilerParams(needs_layout_passes=False))
    def k(m_hbm, rows_hbm, z1_hbm, agg_hbm, idx_v, bm_v, acc_sh, rsem):
        c = lax.axis_index("c")
        s = lax.axis_index("s")
        wid = s * NC + c
        r0 = s * RPT
        pltpu.sync_copy(rows_hbm.at[wid], idx_v)

        @pl.when(s < NS - 1)
        def _():
            pltpu.sync_copy(z1_hbm.at[pl.ds(r0, RPT)], acc_sh.at[pl.ds(r0, RPT)])

        @pl.when(s == NS - 1)
        def _():
            pltpu.sync_copy(z1_hbm.at[pl.ds((NS - 1) * RPT, RPT_LAST)],
                            acc_sh.at[pl.ds((NS - 1) * RPT, RPT_LAST)])

        plsc.subcore_barrier()
        wbase = wid * EPW
        pltpu.async_copy(m_hbm.at[pl.ds(wbase, CH)], bm_v.at[0], rsem)

        def body(i, carry):
            p = lax.rem(i, 2)
            pn = lax.rem(i + 1, 2)
            base = wbase + i * CH

            @pl.when(i + 1 < NCH)
            def _():
                pltpu.async_copy(m_hbm.at[pl.ds(base + CH, CH)],
                                 bm_v.at[pn], rsem)

            pltpu.make_async_copy(m_hbm.at[pl.ds(base, CH)], bm_v.at[p],
                                  rsem).wait()
            pltpu.sync_copy(bm_v.at[p], acc_sh.at[idx_v.at[i]], add=True)
            return carry

        lax.fori_loop(0, NCH, body, 0)
        plsc.subcore_barrier()

        @pl.when(s < NS - 1)
        def _():
            pltpu.sync_copy(acc_sh.at[pl.ds(r0, RPT)],
                            agg_hbm.at[c, pl.ds(r0, RPT)])

        @pl.when(s == NS - 1)
        def _():
            pltpu.sync_copy(acc_sh.at[pl.ds((NS - 1) * RPT, RPT_LAST)],
                            agg_hbm.at[c, pl.ds((NS - 1) * RPT, RPT_LAST)])

    return k(m, rows3, z1)


def _sc_scatter_coord(cmsg, rows3, z2):
    out_type = jax.ShapeDtypeStruct((NW * NNCP_PAD,), jnp.float32)
    scratch = [pltpu.VMEM((NCH, CH), jnp.int32),
               pltpu.VMEM((CH * CP,), jnp.float32),
               pltpu.VMEM((NN * CP,), jnp.float32)]

    @functools.partial(
        pl.kernel, out_type=out_type, mesh=_sc_mesh(),
        scratch_types=tuple(scratch),
        compiler_params=pltpu.CompilerParams(needs_layout_passes=False))
    def k(cm_hbm, rows_hbm, z2_hbm, cs_hbm, idx_v, bc_v, cpriv_v):
        c = lax.axis_index("c")
        s = lax.axis_index("s")
        wid = s * NC + c
        lane = lax.iota(jnp.int32, 16)
        pltpu.sync_copy(rows_hbm.at[wid], idx_v)
        pltpu.sync_copy(z2_hbm, cpriv_v)
        wbase = wid * EPW

        def body(i, carry):
            base = wbase + i * CH
            pltpu.sync_copy(cm_hbm.at[pl.ds(base * CP, CH * CP)], bc_v)
            ones = jnp.full((16,), 1.0, jnp.float32)
            for e in range(0, CH, 16):
                rv = idx_v[i, pl.ds(e, 16)] * CP
                row = (lane + e) * CP
                for d in range(3):
                    v = plsc.load_gather(bc_v, [row + d])
                    plsc.addupdate_scatter(cpriv_v, [rv + d], v)
                plsc.addupdate_scatter(cpriv_v, [rv + 3], ones)
            return carry

        lax.fori_loop(0, NCH, body, 0)
        pltpu.sync_copy(cpriv_v, cs_hbm.at[pl.ds(wid * NNCP_PAD, NNCP)])

    return k(cmsg, rows3, z2)


def _full(arr):
    return pl.BlockSpec(arr.shape, lambda *_: tuple(0 for _ in arr.shape))


def _silu(x):
    return x * jax.nn.sigmoid(x)


def _tc_edge(hr, hc, cdm, ew, with_coord, block=2000):
    (w0a, w0b, w0r, b0, w1, b1, wat_t, bat, wc, bc, ww_t) = ew
    ne = hr.shape[0]
    grid = (ne // block,)

    def body(*refs):
        if with_coord:
            (hr_r, hc_r, cd_r, w0a_r, w0b_r, w0r_r, b0_r, w1_r, b1_r,
             wat_r, bat_r, wc_r, bc_r, ww_r, m_r, cm_r) = refs
        else:
            (hr_r, hc_r, cd_r, w0a_r, w0b_r, w0r_r, b0_r, w1_r, b1_r,
             wat_r, bat_r, m_r) = refs
        cd = cd_r[...]
        rad = (cd[:, 0:1] * cd[:, 0:1] + cd[:, 1:2] * cd[:, 1:2]
               + cd[:, 2:3] * cd[:, 2:3])
        e0 = (jnp.dot(hr_r[...], w0a_r[...], preferred_element_type=jnp.float32)
              + jnp.dot(hc_r[...], w0b_r[...],
                        preferred_element_type=jnp.float32)
              + rad * w0r_r[...] + b0_r[...])
        e0 = _silu(e0)
        e1 = _silu(jnp.dot(e0, w1_r[...], preferred_element_type=jnp.float32)
                   + b1_r[...])
        att = jax.nn.sigmoid(
            jnp.sum(e1 * wat_r[...], axis=1, keepdims=True) + bat_r[...])
        m = e1 * att
        m_r[...] = m
        if with_coord:
            cmi = _silu(jnp.dot(m, wc_r[...], preferred_element_type=jnp.float32)
                        + bc_r[...])
            cw = jnp.tanh(jnp.sum(cmi * ww_r[...], axis=1, keepdims=True))
            cdn = cd / (jnp.sqrt(rad) + EPS)
            cm_r[...] = cdn * cw

    ins = [hr, hc, cdm, w0a, w0b, w0r, b0, w1, b1, wat_t, bat]
    in_specs = [pl.BlockSpec((block, DH), lambda i: (i, 0)),
                pl.BlockSpec((block, DH), lambda i: (i, 0)),
                pl.BlockSpec((block, CP), lambda i: (i, 0))] + \
               [_full(a) for a in (w0a, w0b, w0r, b0, w1, b1, wat_t, bat)]
    out_shape = [jax.ShapeDtypeStruct((ne, DH), jnp.float32)]
    out_specs = [pl.BlockSpec((block, DH), lambda i: (i, 0))]
    if with_coord:
        ins += [wc, bc, ww_t]
        in_specs += [_full(a) for a in (wc, bc, ww_t)]
        out_shape += [jax.ShapeDtypeStruct((ne, CP), jnp.float32)]
        out_specs += [pl.BlockSpec((block, CP), lambda i: (i, 0))]
    out = pl.pallas_call(body, grid=grid, in_specs=in_specs,
                         out_specs=out_specs, out_shape=out_shape)(*ins)
    return out if with_coord else (out[0],)


def _tc_reduce_nw(cs_flat, block=4096):
    x = cs_flat.reshape(NW, NNCP_PAD)
    grid = (NNCP_PAD // block,)

    def body(x_r, o_r):
        o_r[...] = jnp.sum(x_r[...], axis=0)

    out = pl.pallas_call(
        body, grid=grid,
        in_specs=[pl.BlockSpec((NW, block), lambda i: (0, i))],
        out_specs=pl.BlockSpec((block,), lambda i: (i,)),
        out_shape=jax.ShapeDtypeStruct((NNCP_PAD,), jnp.float32))(x)
    return out[:NNCP].reshape(NN, CP)


def _tc_node(h, agg_p, nw, with_coord=False, cs=None, coordp=None, block=2000,
             emit16=False):
    wh, wa, b0, w1, b1 = nw
    n = h.shape[0]
    npart = agg_p.shape[0]
    grid = (n // block,)

    def body(*refs):
        mask3 = (lax.broadcasted_iota(jnp.int32, (1, CP), 1) < 3
                 ).astype(jnp.float32)
        it = iter(refs)
        h_r = next(it)
        ag_r = next(it)
        wh_r = next(it)
        wa_r = next(it)
        b0_r = next(it)
        w1_r = next(it)
        b1_r = next(it)
        cs_r = next(it) if with_coord else None
        cp_r = next(it) if with_coord else None
        hn_r = next(it)
        h16_r = next(it) if emit16 else None
        cn_r = next(it) if with_coord else None
        agg = ag_r[0]
        for p in range(1, npart):
            agg = agg + ag_r[p]
        h = h_r[...]
        t = _silu(jnp.dot(h, wh_r[...], preferred_element_type=jnp.float32)
                  + jnp.dot(agg, wa_r[...], preferred_element_type=jnp.float32)
                  + b0_r[...])
        out = jnp.dot(t, w1_r[...], preferred_element_type=jnp.float32) + b1_r[...]
        hn = h + out
        hn_r[...] = hn
        if emit16:
            h16_r[...] = hn.astype(jnp.bfloat16)
        if with_coord:
            csv = cs_r[...]
            cnt = jnp.maximum(csv[:, 3:4], 1.0)
            cn_r[...] = cp_r[...] + (csv / cnt) * mask3

    ins = [h, agg_p, wh, wa, b0, w1, b1]
    in_specs = [pl.BlockSpec((block, DH), lambda i: (i, 0)),
                pl.BlockSpec((npart, block, DH), lambda i: (0, i, 0))] + \
               [_full(a) for a in (wh, wa, b0, w1, b1)]
    if with_coord:
        ins += [cs, coordp]
        in_specs += [pl.BlockSpec((block, CP), lambda i: (i, 0)),
                     pl.BlockSpec((block, CP), lambda i: (i, 0))]
    out_shape = [jax.ShapeDtypeStruct((n, DH), jnp.float32)]
    out_specs = [pl.BlockSpec((block, DH), lambda i: (i, 0))]
    if emit16:
        out_shape += [jax.ShapeDtypeStruct((n, DH), jnp.bfloat16)]
        out_specs += [pl.BlockSpec((block, DH), lambda i: (i, 0))]
    if with_coord:
        out_shape += [jax.ShapeDtypeStruct((n, CP), jnp.float32)]
        out_specs += [pl.BlockSpec((block, CP), lambda i: (i, 0))]
    out = pl.pallas_call(body, grid=grid, in_specs=in_specs,
                         out_specs=out_specs, out_shape=out_shape)(*ins)
    return out


def _tc_lin(x, w, b, block=None, emit16=False):
    m, kdim = x.shape
    dout = w.shape[1]
    if block is None:
        block = m if m <= 2000 else 2000
    grid = (m // block,)
    b2 = b.reshape(1, dout)

    def body(*refs):
        if emit16:
            x_r, w_r, b_r, y_r, y16_r = refs
        else:
            x_r, w_r, b_r, y_r = refs
        y = (jnp.dot(x_r[...], w_r[...], preferred_element_type=jnp.float32)
             + b_r[...])
        y_r[...] = y
        if emit16:
            y16_r[...] = y.astype(jnp.bfloat16)

    out_shape = [jax.ShapeDtypeStruct((m, dout), jnp.float32)]
    out_specs = [pl.BlockSpec((block, dout), lambda i: (i, 0))]
    if emit16:
        out_shape += [jax.ShapeDtypeStruct((m, dout), jnp.bfloat16)]
        out_specs += [pl.BlockSpec((block, dout), lambda i: (i, 0))]
    out = pl.pallas_call(
        body, grid=grid,
        in_specs=[pl.BlockSpec((block, kdim), lambda i: (i, 0)),
                  _full(w), _full(b2)],
        out_specs=out_specs,
        out_shape=out_shape)(x, w, b2)
    return out if emit16 else out[0]


def _tc_pool(x, bidx, block=2000):
    n, f = x.shape
    grid = (n // block,)
    last = n // block - 1

    def body(x_r, b_r, o_r):
        i = pl.program_id(0)
        oh = (b_r[...] == lax.broadcasted_iota(jnp.int32, (1, BB), 1)
              ).astype(jnp.float32)
        part = lax.dot_general(oh, x_r[...], (((0,), (0,)), ((), ())),
                               preferred_element_type=jnp.float32)

        @pl.when(i == 0)
        def _():
            o_r[...] = part

        @pl.when(i > 0)
        def _():
            o_r[...] = o_r[...] + part

        @pl.when(i == last)
        def _():
            s = o_r[...]
            o_r[...] = s / jnp.maximum(s[:, f - 1:f], 1.0)

    return pl.pallas_call(
        body, grid=grid,
        in_specs=[pl.BlockSpec((block, f), lambda i: (i, 0)),
                  pl.BlockSpec((block, 1), lambda i: (i, 0))],
        out_specs=pl.BlockSpec((BB, f), lambda i: (0, 0)),
        out_shape=jax.ShapeDtypeStruct((BB, f), jnp.float32))(x, bidx)


def _tc_small(h64, coords, rmat, cmat, cnt2, ew, with_coord,
              node_w=None):
    (w0a, w0b, w0r, b0, w1, b1, wat_t, bat, wc, bc, ww_t) = ew
    node_mlp = node_w is not None

    def body(*refs):
        mask3 = (lax.broadcasted_iota(jnp.int32, (1, CP), 1) < 3
                 ).astype(jnp.float32)
        it = iter(refs)
        h_r = next(it); cs_r = next(it); r_r = next(it); c_r = next(it)
        cnt_r = next(it)
        w0a_r = next(it); w0b_r = next(it); w0r_r = next(it); b0_r = next(it)
        w1_r = next(it); b1_r = next(it); wat_r = next(it); bat_r = next(it)
        if with_coord:
            wc_r = next(it); bc_r = next(it); ww_r = next(it)
        if node_mlp:
            nwh_r = next(it); nwa_r = next(it); nb0_r = next(it)
            nw1_r = next(it); nb1_r = next(it)
        o1_r = next(it)
        if with_coord:
            o2_r = next(it)
        h = h_r[...]
        cso = cs_r[...]
        rm = r_r[...]
        cm = c_r[...]
        hr = jnp.dot(rm, h, preferred_element_type=jnp.float32)
        hc = jnp.dot(cm, h, preferred_element_type=jnp.float32)
        cr = jnp.dot(rm, cso, preferred_element_type=jnp.float32)
        cc = jnp.dot(cm, cso, preferred_element_type=jnp.float32)
        cd = cr - cc
        rad = (cd[:, 0:1] * cd[:, 0:1] + cd[:, 1:2] * cd[:, 1:2]
               + cd[:, 2:3] * cd[:, 2:3])
        e0 = _silu(jnp.dot(hr, w0a_r[...], preferred_element_type=jnp.float32)
                   + jnp.dot(hc, w0b_r[...], preferred_element_type=jnp.float32)
                   + rad * w0r_r[...] + b0_r[...])
        e1 = _silu(jnp.dot(e0, w1_r[...], preferred_element_type=jnp.float32)
                   + b1_r[...])
        att = jax.nn.sigmoid(
            jnp.sum(e1 * wat_r[...], axis=1, keepdims=True) + bat_r[...])
        m = e1 * att
        agg = lax.dot_general(rm, m, (((0,), (0,)), ((), ())),
                              preferred_element_type=jnp.float32)
        if with_coord:
            cmi = _silu(jnp.dot(m, wc_r[...], preferred_element_type=jnp.float32)
                        + bc_r[...])
            cw = jnp.tanh(jnp.sum(cmi * ww_r[...], axis=1, keepdims=True))
            cdn = cd / (jnp.sqrt(rad) + EPS)
            cmsg = cdn * cw
            csum = lax.dot_general(rm, cmsg, (((0,), (0,)), ((), ())),
                                   preferred_element_type=jnp.float32)
            o2_r[...] = cso + (csum / cnt_r[...]) * mask3
        if node_mlp:
            t = _silu(jnp.dot(h, nwh_r[...], preferred_element_type=jnp.float32)
                      + jnp.dot(agg, nwa_r[...], preferred_element_type=jnp.float32)
                      + nb0_r[...])
            o1_r[...] = h + (jnp.dot(t, nw1_r[...],
                                     preferred_element_type=jnp.float32)
                             + nb1_r[...])
        else:
            o1_r[...] = agg

    ins = [h64, coords, rmat, cmat, cnt2,
           w0a, w0b, w0r, b0, w1, b1, wat_t, bat]
    if with_coord:
        ins += [wc, bc, ww_t]
    if node_mlp:
        ins += list(node_w)
    in_specs = [_full(a) for a in ins]
    out_shape = [jax.ShapeDtypeStruct((BB, DH), jnp.float32)]
    out_specs = [_full(jnp.zeros((BB, DH)))]
    if with_coord:
        out_shape += [jax.ShapeDtypeStruct((BB, CP), jnp.float32)]
        out_specs += [_full(jnp.zeros((BB, CP)))]
    out = pl.pallas_call(body, grid=(1,), in_specs=in_specs,
                         out_specs=out_specs, out_shape=out_shape)(*ins)
    return out


def _tc_head(pool2, out_seq, bn1, bn2, bn3, wf_parts, bf, fbn, wfin, bfin):
    wf_a, wf_b, wf_c, wf_d = wf_parts

    def _bn(x, g, b):
        mu = jnp.mean(x, axis=0, keepdims=True)
        var = jnp.mean((x - mu) * (x - mu), axis=0, keepdims=True)
        return g * (x - mu) / jnp.sqrt(var + 1e-5) + b

    def body(p_r, os_r, g1_r, b1_r, g2_r, b2_r, g3_r, b3_r,
             wfa_r, wfb_r, wfc_r, wfd_r, bf_r, gf_r, bfg_r,
             wfin_r, bfin_r, o_r):
        p = p_r[...]
        pr = jnp.maximum(_bn(p[:, 0:128], g1_r[...], b1_r[...]), 0.0)
        pr2 = jnp.maximum(_bn(p[:, 128:192], g2_r[...], b2_r[...]), 0.0)
        pr3 = jnp.maximum(_bn(p[:, 192:224], g3_r[...], b3_r[...]), 0.0)
        ps = jnp.maximum(_bn(os_r[...], g2_r[...], b2_r[...]), 0.0)
        x = (jnp.dot(pr, wfa_r[...], preferred_element_type=jnp.float32)
             + jnp.dot(ps, wfb_r[...], preferred_element_type=jnp.float32)
             + jnp.dot(pr2, wfc_r[...], preferred_element_type=jnp.float32)
             + jnp.dot(pr3, wfd_r[...], preferred_element_type=jnp.float32)
             + bf_r[...])
        x = jnp.maximum(_bn(x, gf_r[...], bfg_r[...]), 0.0)
        x = jnp.dot(x, wfin_r[...], preferred_element_type=jnp.float32) + bfin_r[...]
        o_r[...] = jax.nn.sigmoid(x)

    ins = [pool2, out_seq,
           bn1[0].reshape(1, -1), bn1[1].reshape(1, -1),
           bn2[0].reshape(1, -1), bn2[1].reshape(1, -1),
           bn3[0].reshape(1, -1), bn3[1].reshape(1, -1),
           wf_a, wf_b, wf_c, wf_d, bf.reshape(1, -1),
           fbn[0].reshape(1, -1), fbn[1].reshape(1, -1),
           wfin, bfin.reshape(1, -1)]
    return pl.pallas_call(
        body, grid=(1,),
        in_specs=[_full(a) for a in ins],
        out_specs=_full(jnp.zeros((BB, 128))),
        out_shape=jax.ShapeDtypeStruct((BB, 128), jnp.float32))(*ins)


def _edge_weights(gp):
    w0 = gp['edge0'][0]
    return (w0[:DH], w0[DH:2 * DH], w0[2 * DH:2 * DH + 1],
            gp['edge0'][1].reshape(1, DH),
            gp['edge1'][0], gp['edge1'][1].reshape(1, DH),
            gp['att'][0].reshape(1, DH), gp['att'][1].reshape(1, 1),
            gp['coord0'][0], gp['coord0'][1].reshape(1, DH),
            gp['coordw'][0].reshape(1, DH))


def _node_weights(gp):
    wn0 = gp['node0'][0]
    return (wn0[:DH], wn0[DH:], gp['node0'][1].reshape(1, DH),
            gp['node1'][0], gp['node1'][1].reshape(1, DH))


def _big_gcl(h, h16, coordp, rows, cols, gp, z1, z2, with_coord,
             cd_pre=None, emit16=False):
    ew = _edge_weights(gp)
    nw = _node_weights(gp)
    if cd_pre is None:
        hr, hc, cd = _sc_gather(h16, coordp.reshape(-1), rows, cols,
                                with_cd=True)
        cd = cd.reshape(NE, CP)
    else:
        hr, hc = _sc_gather(h16, coordp, rows, cols, with_cd=False)
        cd = cd_pre
    e = _tc_edge(hr, hc, cd, ew, with_coord)
    if with_coord:
        m, cmsg = e
        agg_p, = _sc_scatter(m, rows, z1)
        cs_flat = _sc_scatter_coord(cmsg.reshape(-1), rows, z2)
        cs = _tc_reduce_nw(cs_flat)
        outs = _tc_node(h, agg_p, nw, True, cs, coordp, emit16=emit16)
        if emit16:
            hn, hn16, cn = outs
        else:
            (hn, cn), hn16 = outs, None
        return hn, hn16, cn, cd
    m, = e
    agg_p, = _sc_scatter(m, rows, z1)
    outs = _tc_node(h, agg_p, nw, False, emit16=emit16)
    if emit16:
        hn, hn16 = outs
    else:
        (hn,), hn16 = outs, None
    return hn, hn16, None, cd


def kernel(x_res, x_emb_seq, x_pos, edge_index, x_batch, params):
    rows = edge_index[0].astype(jnp.int32).reshape(NW, NCH, CH)
    cols = edge_index[1].astype(jnp.int32).reshape(NW, NCH, CH)
    coordp0 = jnp.pad(x_pos.astype(jnp.float32), ((0, 0), (0, CP - 3)))
    z1 = jnp.zeros((NN, DH), jnp.float32)
    z2 = jnp.zeros((NN * CP,), jnp.float32)
    bidx = x_batch.astype(jnp.int32).reshape(NN, 1)
    rmat = jnp.asarray(_R_NP)
    cmat = jnp.asarray(_C_NP)
    cnt2 = jnp.asarray(_CNT2_NP)
    ones_n = jnp.ones((NN, 1), jnp.float32)

    pooled = _tc_pool(jnp.concatenate([x_pos, ones_n], axis=1), bidx)
    pooled_cs = jnp.pad(pooled[:, :3], ((0, 0), (0, CP - 3)))

    p1 = params['egnn1']
    h = _tc_lin(x_res, p1['emb_in'][0], p1['emb_in'][1])
    h, _, c1, cd0 = _big_gcl(h, h, coordp0, rows, cols, p1['gcl'][0],
                             z1, z2, True)
    h, _, _, _ = _big_gcl(h, h, c1, rows, cols, p1['gcl'][1], z1, z2, False)
    out_res = _tc_lin(h, p1['emb_out'][0], p1['emb_out'][1])

    p2 = params['egnn2']
    h = _tc_lin(out_res, p2['emb_in'][0], p2['emb_in'][1])
    h, _, c1, _ = _big_gcl(h, h, coordp0, rows, cols, p2['gcl'][0],
                           z1, z2, True, cd_pre=cd0)
    h, _, _, _ = _big_gcl(h, h, c1, rows, cols, p2['gcl'][1], z1, z2, False)
    out_res2 = _tc_lin(h, p2['emb_out'][0], p2['emb_out'][1])

    p4 = params['egnn4']
    h4 = _tc_lin(out_res2, p4['emb_in'][0], p4['emb_in'][1])
    ew40 = _edge_weights(p4['gcl'][0])
    agg64, cs_new = _tc_small(h4[:BB], pooled_cs, rmat, cmat, cnt2, ew40, True)
    agg_full = jnp.pad(agg64, ((0, NN - BB), (0, 0)))[None]
    h4, = _tc_node(h4, agg_full, _node_weights(p4['gcl'][0]), False)
    ew41 = _edge_weights(p4['gcl'][1])
    agg64b, = _tc_small(h4[:BB], cs_new, rmat, cmat, cnt2, ew41, False)
    agg_full = jnp.pad(agg64b, ((0, NN - BB), (0, 0)))[None]
    h4, = _tc_node(h4, agg_full, _node_weights(p4['gcl'][1]), False)
    out_res3 = _tc_lin(h4, p4['emb_out'][0], p4['emb_out'][1])

    p3 = params['egnn3']
    h3 = _tc_lin(x_emb_seq, p3['emb_in'][0], p3['emb_in'][1])
    ew30 = _edge_weights(p3['gcl'][0])
    h3, cs3 = _tc_small(h3, pooled_cs, rmat, cmat, cnt2, ew30, True,
                        node_w=_node_weights(p3['gcl'][0]))
    ew31 = _edge_weights(p3['gcl'][1])
    h3, = _tc_small(h3, cs3, rmat, cmat, cnt2, ew31, False,
                    node_w=_node_weights(p3['gcl'][1]))
    out_seq = _tc_lin(h3, p3['emb_out'][0], p3['emb_out'][1])

    pool2 = _tc_pool(jnp.concatenate([out_res, out_res2, out_res3, ones_n],
                                     axis=1), bidx)

    wf = params['fc1'][0]
    wf_parts = (wf[0:128], wf[128:192], wf[192:256], wf[256:288])
    return _tc_head(pool2, out_seq, params['bn1'], params['bn2'], params['bn3'],
                    wf_parts, params['fc1'][1], params['fc1_bn'],
                    params['final'][0], params['final'][1])
